# baseline (speedup 1.0000x reference)
import functools

import jax
import jax.numpy as jnp
from jax import lax
from jax.experimental import pallas as pl
from jax.experimental.pallas import tpu as pltpu
from jax.experimental.pallas import tpu_sc as plsc

N = 100000
D = 128
M = 4096

ROWS = 25088
NBLK = 4
NPAD = NBLK * ROWS

NC = 2
NS = 16
NW = NC * NS
CHUNK = NPAD // NW
VREGS = CHUNK // 16
TAIL_VALID = N - (NW - 1) * CHUNK


def _matvec_body(e_ref, w_ref, b_ref, y_ref):
    g = pl.program_id(0)
    y = lax.dot_general(
        w_ref[...], e_ref[...],
        dimension_numbers=(((1,), (1,)), ((), ())),
        preferred_element_type=jnp.float32,
    ) + b_ref[0]
    col = g * ROWS + lax.broadcasted_iota(jnp.int32, (1, ROWS), 1)
    y_ref[0, :, :] = jnp.where(col < N, y, 0.0)


def _matvec(energy, w_row, b):
    return pl.pallas_call(
        _matvec_body,
        grid=(NBLK,),
        in_specs=[
            pl.BlockSpec((ROWS, D), lambda g: (g, 0)),
            pl.BlockSpec((1, D), lambda g: (0, 0)),
            pl.BlockSpec(memory_space=pltpu.SMEM),
        ],
        out_specs=pl.BlockSpec((1, 1, ROWS), lambda g: (g, 0, 0)),
        out_shape=jax.ShapeDtypeStruct((NBLK, 1, ROWS), jnp.float32),
    )(energy, w_row, b)


def _seg_body(y_hbm, batch_hbm, out_hbm, y_v, idx_v, acc_v):
    c = lax.axis_index("c")
    s = lax.axis_index("s")
    wid = s * NC + c
    base = wid * CHUNK

    pltpu.sync_copy(y_hbm.at[pl.ds(base, CHUNK)], y_v)

    @pl.when(wid < NW - 1)
    def _():
        pltpu.sync_copy(batch_hbm.at[pl.ds(base, CHUNK)], idx_v)

    @pl.when(wid == NW - 1)
    def _():
        pltpu.sync_copy(
            batch_hbm.at[pl.ds(base, TAIL_VALID)], idx_v.at[pl.ds(0, TAIL_VALID)]
        )
        def zb(j, carry):
            idx_v[pl.ds(TAIL_VALID + j * 16, 16)] = jnp.zeros((16,), jnp.int32)
            return carry
        lax.fori_loop(0, (CHUNK - TAIL_VALID) // 16, zb, 0)

    def zero_acc(j, carry):
        acc_v[pl.ds(j * 16, 16)] = jnp.zeros((16,), jnp.float32)
        return carry
    lax.fori_loop(0, M // 16, zero_acc, 0)

    def body(j, carry):
        v = y_v[pl.ds(j * 16, 16)]
        ix = idx_v[pl.ds(j * 16, 16)]
        plsc.addupdate_scatter(acc_v, [ix], v)
        return carry
    lax.fori_loop(0, VREGS, body, 0)

    pltpu.sync_copy(acc_v, out_hbm.at[wid])


def _segment_partials(y_flat, batch):
    mesh = plsc.VectorSubcoreMesh(core_axis_name="c", subcore_axis_name="s")
    f = functools.partial(
        pl.kernel,
        out_type=jax.ShapeDtypeStruct((NW, M), jnp.float32),
        mesh=mesh,
        scratch_types=[
            pltpu.VMEM((CHUNK,), jnp.float32),
            pltpu.VMEM((CHUNK,), jnp.int32),
            pltpu.VMEM((M,), jnp.float32),
        ],
        compiler_params=pltpu.CompilerParams(needs_layout_passes=False),
    )(_seg_body)
    return f(y_flat, batch)


def _reduce_body(p_ref, o_ref):
    o_ref[...] = jnp.sum(p_ref[...], axis=0)


def _reduce(partials):
    return pl.pallas_call(
        _reduce_body,
        out_shape=jax.ShapeDtypeStruct((M,), jnp.float32),
    )(partials)


def kernel(energy, batch, W, b):
    w_row = W.reshape(1, D)
    batch32 = batch.astype(jnp.int32)
    y = _matvec(energy, w_row, b)
    y_flat = y.reshape(NPAD)
    partials = _segment_partials(y_flat, batch32)
    return _reduce(partials)


# speedup vs baseline: 1.0154x; 1.0154x over previous
import functools

import jax
import jax.numpy as jnp
from jax import lax
from jax.experimental import pallas as pl
from jax.experimental.pallas import tpu as pltpu
from jax.experimental.pallas import tpu_sc as plsc

N = 100000
D = 128
M = 4096

ROWS = 14336
NBLK = 7
NPAD = NBLK * ROWS

NS = 16
CHUNK = NPAD // NS
VREGS = CHUNK // 16
TAIL_VALID = N - (NS - 1) * CHUNK
SEG = M // NS


def _matvec_body(e_ref, w_ref, b_ref, y_ref):
    g = pl.program_id(0)
    y = lax.dot_general(
        w_ref[...], e_ref[...],
        dimension_numbers=(((1,), (1,)), ((), ())),
        preferred_element_type=jnp.float32,
    ) + b_ref[0]
    col = g * ROWS + lax.broadcasted_iota(jnp.int32, (1, ROWS), 1)
    y_ref[0, :, :] = jnp.where(col < N, y, 0.0)


def _matvec(energy, w_row, b):
    return pl.pallas_call(
        _matvec_body,
        grid=(NBLK,),
        in_specs=[
            pl.BlockSpec((ROWS, D), lambda g: (g, 0)),
            pl.BlockSpec((1, D), lambda g: (0, 0)),
            pl.BlockSpec(memory_space=pltpu.SMEM),
        ],
        out_specs=pl.BlockSpec((1, 1, ROWS), lambda g: (g, 0, 0)),
        out_shape=jax.ShapeDtypeStruct((NBLK, 1, ROWS), jnp.float32),
    )(energy, w_row, b)


def _seg_body(y_hbm, batch_hbm, out_hbm, y_v, idx_v, acc_v, slab_v, o_v, spmem):
    s = lax.axis_index("s")
    base = s * CHUNK

    pltpu.sync_copy(y_hbm.at[pl.ds(base, CHUNK)], y_v)

    @pl.when(s < NS - 1)
    def _():
        pltpu.sync_copy(batch_hbm.at[pl.ds(base, CHUNK)], idx_v)

    @pl.when(s == NS - 1)
    def _():
        pltpu.sync_copy(
            batch_hbm.at[pl.ds(base, TAIL_VALID)], idx_v.at[pl.ds(0, TAIL_VALID)]
        )
        def zb(j, carry):
            idx_v[pl.ds(TAIL_VALID + j * 16, 16)] = jnp.zeros((16,), jnp.int32)
            return carry
        lax.fori_loop(0, (CHUNK - TAIL_VALID) // 16, zb, 0)

    zero16 = jnp.zeros((16,), jnp.float32)

    def zero_acc(j, carry):
        for u in range(8):
            acc_v[pl.ds(j * 128 + u * 16, 16)] = zero16
        return carry
    lax.fori_loop(0, M // 128, zero_acc, 0)

    def body(j, carry):
        for u in range(4):
            off = j * 64 + u * 16
            plsc.addupdate_scatter(
                acc_v, [idx_v[pl.ds(off, 16)]], y_v[pl.ds(off, 16)]
            )
        return carry
    lax.fori_loop(0, VREGS // 4, body, 0)

    pltpu.sync_copy(acc_v, spmem.at[s])
    plsc.subcore_barrier()
    pltpu.sync_copy(spmem.at[:, pl.ds(s * SEG, SEG)], slab_v)

    def red(t, carry):
        tot = slab_v[0, pl.ds(t * 16, 16)]
        for k in range(1, NS):
            tot = tot + slab_v[k, pl.ds(t * 16, 16)]
        o_v[pl.ds(t * 16, 16)] = tot
        return carry
    lax.fori_loop(0, SEG // 16, red, 0)

    pltpu.sync_copy(o_v, out_hbm.at[pl.ds(s * SEG, SEG)])


def _segment_sum(y_flat, batch):
    mesh = plsc.VectorSubcoreMesh(
        core_axis_name="c", subcore_axis_name="s", num_cores=1
    )
    f = functools.partial(
        pl.kernel,
        out_type=jax.ShapeDtypeStruct((M,), jnp.float32),
        mesh=mesh,
        scratch_types=[
            pltpu.VMEM((CHUNK,), jnp.float32),
            pltpu.VMEM((CHUNK,), jnp.int32),
            pltpu.VMEM((M,), jnp.float32),
            pltpu.VMEM((NS, SEG), jnp.float32),
            pltpu.VMEM((SEG,), jnp.float32),
            pltpu.VMEM_SHARED((NS, M), jnp.float32),
        ],
        compiler_params=pltpu.CompilerParams(needs_layout_passes=False),
    )(_seg_body)
    return f(y_flat, batch)


def kernel(energy, batch, W, b):
    w_row = W.reshape(1, D)
    batch32 = batch.astype(jnp.int32)
    y = _matvec(energy, w_row, b)
    y_flat = y.reshape(NPAD)
    return _segment_sum(y_flat, batch32)


# speedup vs baseline: 1.0393x; 1.0235x over previous
import functools

import jax
import jax.numpy as jnp
from jax import lax
from jax.experimental import pallas as pl
from jax.experimental.pallas import tpu as pltpu
from jax.experimental.pallas import tpu_sc as plsc

N = 100000
D = 128
M = 4096

ROWS = 12544
NBLK = 4
HALF = NBLK * ROWS
NPAD = 2 * HALF

NS = 16
CHUNK = HALF // NS
VREGS = CHUNK // 16
TAIL_VALID = N - HALF - (NS - 1) * CHUNK
SEG = M // NS


def _make_matvec_body(off):
    def body(e_ref, w_ref, b_ref, y_ref):
        g = pl.program_id(0)
        y = lax.dot_general(
            w_ref[...], e_ref[...],
            dimension_numbers=(((1,), (1,)), ((), ())),
            preferred_element_type=jnp.float32,
        ) + b_ref[0]
        col = (g + off) * ROWS + lax.broadcasted_iota(jnp.int32, (1, ROWS), 1)
        y_ref[0, :, :] = jnp.where(col < N, y, 0.0)
    return body


def _matvec(energy, w_row, b, off):
    return pl.pallas_call(
        _make_matvec_body(off),
        grid=(NBLK,),
        in_specs=[
            pl.BlockSpec((ROWS, D), lambda g: (g + off, 0)),
            pl.BlockSpec((1, D), lambda g: (0, 0)),
            pl.BlockSpec(memory_space=pltpu.SMEM),
        ],
        out_specs=pl.BlockSpec((1, 1, ROWS), lambda g: (g, 0, 0)),
        out_shape=jax.ShapeDtypeStruct((NBLK, 1, ROWS), jnp.float32),
    )(energy, w_row, b)


def _make_seg_body(off_elems, tail_valid, has_prev):
    def body(*args):
        if has_prev:
            (y_hbm, batch_hbm, prev_hbm, out_hbm,
             y_v, idx_v, acc_v, slab_v, o_v, prev_v, sem_y, spmem) = args
        else:
            (y_hbm, batch_hbm, out_hbm,
             y_v, idx_v, acc_v, slab_v, o_v, prev_v, sem_y, spmem) = args
        s = lax.axis_index("s")
        base = s * CHUNK
        gbase = off_elems + base

        cp_y = pltpu.make_async_copy(y_hbm.at[pl.ds(base, CHUNK)], y_v, sem_y)
        cp_y.start()

        @pl.when(s < NS - 1)
        def _():
            pltpu.sync_copy(batch_hbm.at[pl.ds(gbase, CHUNK)], idx_v)

        @pl.when(s == NS - 1)
        def _():
            if tail_valid == CHUNK:
                pltpu.sync_copy(batch_hbm.at[pl.ds(gbase, CHUNK)], idx_v)
            else:
                pltpu.sync_copy(
                    batch_hbm.at[pl.ds(gbase, tail_valid)],
                    idx_v.at[pl.ds(0, tail_valid)],
                )
                def zb(j, carry):
                    idx_v[pl.ds(tail_valid + j * 16, 16)] = jnp.zeros(
                        (16,), jnp.int32
                    )
                    return carry
                lax.fori_loop(0, (CHUNK - tail_valid) // 16, zb, 0)

        zero16 = jnp.zeros((16,), jnp.float32)

        def zero_acc(j, carry):
            for u in range(8):
                acc_v[pl.ds(j * 128 + u * 16, 16)] = zero16
            return carry
        lax.fori_loop(0, M // 128, zero_acc, 0)

        cp_y.wait()

        def scat(j, carry):
            for u in range(4):
                off = j * 64 + u * 16
                plsc.addupdate_scatter(
                    acc_v, [idx_v[pl.ds(off, 16)]], y_v[pl.ds(off, 16)]
                )
            return carry
        lax.fori_loop(0, VREGS // 4, scat, 0)

        pltpu.sync_copy(acc_v, spmem.at[s])
        if has_prev:
            pltpu.sync_copy(prev_hbm.at[pl.ds(s * SEG, SEG)], prev_v)
        plsc.subcore_barrier()
        pltpu.sync_copy(spmem.at[:, pl.ds(s * SEG, SEG)], slab_v)

        def red(t, carry):
            tot = slab_v[0, pl.ds(t * 16, 16)]
            for k in range(1, NS):
                tot = tot + slab_v[k, pl.ds(t * 16, 16)]
            if has_prev:
                tot = tot + prev_v[pl.ds(t * 16, 16)]
            o_v[pl.ds(t * 16, 16)] = tot
            return carry
        lax.fori_loop(0, SEG // 16, red, 0)

        pltpu.sync_copy(o_v, out_hbm.at[pl.ds(s * SEG, SEG)])
    return body


def _segment_sum(y_flat, batch, off_elems, tail_valid, prev=None):
    mesh = plsc.VectorSubcoreMesh(
        core_axis_name="c", subcore_axis_name="s", num_cores=1
    )
    f = functools.partial(
        pl.kernel,
        out_type=jax.ShapeDtypeStruct((M,), jnp.float32),
        mesh=mesh,
        scratch_types=[
            pltpu.VMEM((CHUNK,), jnp.float32),
            pltpu.VMEM((CHUNK,), jnp.int32),
            pltpu.VMEM((M,), jnp.float32),
            pltpu.VMEM((NS, SEG), jnp.float32),
            pltpu.VMEM((SEG,), jnp.float32),
            pltpu.VMEM((SEG,), jnp.float32),
            pltpu.SemaphoreType.DMA,
            pltpu.VMEM_SHARED((NS, M), jnp.float32),
        ],
        compiler_params=pltpu.CompilerParams(needs_layout_passes=False),
    )(_make_seg_body(off_elems, tail_valid, prev is not None))
    if prev is None:
        return f(y_flat, batch)
    return f(y_flat, batch, prev)


def kernel(energy, batch, W, b):
    w_row = W.reshape(1, D)
    batch32 = batch.astype(jnp.int32)
    y0 = _matvec(energy, w_row, b, 0).reshape(HALF)
    p0 = _segment_sum(y0, batch32, 0, CHUNK)
    y1 = _matvec(energy, w_row, b, NBLK).reshape(HALF)
    return _segment_sum(y1, batch32, HALF, TAIL_VALID, prev=p0)
